# baseline (speedup 1.0000x reference)
import functools

import jax
import jax.numpy as jnp
from jax import lax
from jax.experimental import pallas as pl
from jax.experimental.pallas import tpu as pltpu
from jax.experimental.pallas import tpu_sc as plsc

NN = 10000
EE = 320000
GG = 64
NW = 32
EPW = EE // NW
CH = 125
NCH = EPW // CH
RPS = 624
DW = 16
NBUF = 8
KPF = 6

_R = 2000
_GRID = NN // _R


def _sc_mesh():
    return plsc.VectorSubcoreMesh(core_axis_name="c", subcore_axis_name="s")


def _striped_copy(s, mk_src, mk_dst):
    base = pl.multiple_of(s * RPS, 8)

    @pl.when(s < 15)
    def _main():
        pltpu.sync_copy(mk_src(base, RPS), mk_dst(base, RPS))

    @pl.when(s == 15)
    def _tail():
        pltpu.sync_copy(mk_src(15 * RPS, NN - 15 * RPS),
                        mk_dst(15 * RPS, NN - 15 * RPS))


@functools.partial(
    pl.kernel,
    out_type=jax.ShapeDtypeStruct((2, NN, DW), jnp.float32),
    mesh=_sc_mesh(),
    scratch_types=[
        pltpu.VMEM((NCH, CH), jnp.int32),
        pltpu.VMEM((CH, DW), jnp.float32),
        pltpu.VMEM_SHARED((NN, DW), jnp.float32),
    ],
    compiler_params=pltpu.CompilerParams(use_tc_tiling_on_sc=False),
)
def _deg_kernel(ei, ones_hbm, zeros_hbm, out, idx_v, ones_v, deg_sh):
    c = lax.axis_index("c")
    s = lax.axis_index("s")
    w = c * 16 + s
    _striped_copy(s, lambda b, n: zeros_hbm.at[pl.ds(b, n)],
                  lambda b, n: deg_sh.at[pl.ds(b, n)])
    pltpu.sync_copy(ones_hbm, ones_v)
    pltpu.sync_copy(ei.at[0, w], idx_v)
    plsc.subcore_barrier()

    def step(j, carry):
        pltpu.sync_copy(ones_v, deg_sh.at[idx_v.at[j]], add=True)
        return carry

    lax.fori_loop(0, NCH, step, 0)
    plsc.subcore_barrier()
    _striped_copy(s, lambda b, n: deg_sh.at[pl.ds(b, n)],
                  lambda b, n: out.at[c, pl.ds(b, n)])


def _make_agg(F):
    @functools.partial(
        pl.kernel,
        out_type=jax.ShapeDtypeStruct((2, NN, F), jnp.float32),
        mesh=_sc_mesh(),
        scratch_types=[
            pltpu.VMEM((NCH, CH), jnp.int32),
            pltpu.VMEM((NCH, CH), jnp.int32),
            pltpu.VMEM((NBUF, CH, F), jnp.float32),
            pltpu.VMEM_SHARED((NN, F), jnp.float32),
            [pltpu.SemaphoreType.DMA] * NBUF,
            [pltpu.SemaphoreType.DMA] * NBUF,
        ],
        compiler_params=pltpu.CompilerParams(use_tc_tiling_on_sc=False),
    )
    def agg(hs, ei, zeros_hbm, out, srci_v, dsti_v, rows_v, agg_sh, gsems,
            ssems):
        c = lax.axis_index("c")
        s = lax.axis_index("s")
        w = c * 16 + s
        _striped_copy(s, lambda b, n: zeros_hbm.at[pl.ds(b, n)],
                      lambda b, n: agg_sh.at[pl.ds(b, n)])
        pltpu.sync_copy(ei.at[0, w], srci_v)
        pltpu.sync_copy(ei.at[1, w], dsti_v)
        plsc.subcore_barrier()

        for b in range(KPF):
            pltpu.async_copy(hs.at[srci_v.at[b]], rows_v.at[b], gsems[b])

        def outer(j0, carry):
            for r in range(NBUF):
                j = j0 * NBUF + r
                nb = (r + KPF) % NBUF
                pltpu.make_async_copy(
                    hs.at[srci_v.at[j]], rows_v.at[r], gsems[r]).wait()
                pltpu.async_copy(rows_v.at[r], agg_sh.at[dsti_v.at[j]],
                                 ssems[r], add=True)
                nxt = j + KPF

                @pl.when(nxt < NCH)
                def _prefetch():
                    @pl.when(j >= NBUF - KPF)
                    def _drain():
                        pltpu.make_async_copy(
                            rows_v.at[nb], agg_sh.at[dsti_v.at[0]],
                            ssems[nb]).wait()

                    pltpu.async_copy(hs.at[srci_v.at[nxt]], rows_v.at[nb],
                                     gsems[nb])
            return carry

        lax.fori_loop(0, NCH // NBUF, outer, 0)
        for b in range(NBUF):
            pltpu.make_async_copy(rows_v.at[b], agg_sh.at[dsti_v.at[0]],
                                  ssems[b]).wait()
        plsc.subcore_barrier()
        _striped_copy(s, lambda b, n: agg_sh.at[pl.ds(b, n)],
                      lambda b, n: out.at[c, pl.ds(b, n)])

    return agg


_agg32 = _make_agg(32)
_agg64 = _make_agg(64)


def _mm1_body(x, w1, h1):
    h1[...] = jnp.dot(x[...], w1[...], preferred_element_type=jnp.float32)


def _mm1(x, w1):
    return pl.pallas_call(
        _mm1_body,
        grid=(_GRID,),
        in_specs=[
            pl.BlockSpec((_R, 128), lambda i: (i, 0)),
            pl.BlockSpec((128, 32), lambda i: (0, 0)),
        ],
        out_specs=pl.BlockSpec((_R, 32), lambda i: (i, 0)),
        out_shape=jax.ShapeDtypeStruct((NN, 32), jnp.float32),
    )(x, w1)


def _dinv_col(degp):
    deg = degp[0, :, 0:1] + degp[1, :, 0:1] + 1.0
    return lax.rsqrt(deg)


def _scale1_body(degp, h1, hs1):
    hs1[...] = h1[...] * _dinv_col(degp)


def _scale1(degp, h1):
    return pl.pallas_call(
        _scale1_body,
        grid=(_GRID,),
        in_specs=[
            pl.BlockSpec((2, _R, DW), lambda i: (0, i, 0)),
            pl.BlockSpec((_R, 32), lambda i: (i, 0)),
        ],
        out_specs=pl.BlockSpec((_R, 32), lambda i: (i, 0)),
        out_shape=jax.ShapeDtypeStruct((NN, 32), jnp.float32),
    )(degp, h1)


def _mid_body(degp, p1, hs1, b1, w2, hs2):
    dinv = _dinv_col(degp)
    t = (p1[0] + p1[1] + hs1[...]) * dinv + b1[...]
    t = jnp.maximum(t, 0.0)
    h = jnp.dot(t, w2[...], preferred_element_type=jnp.float32)
    hs2[...] = h * dinv


def _mid(degp, p1, hs1, b1, w2):
    return pl.pallas_call(
        _mid_body,
        grid=(_GRID,),
        in_specs=[
            pl.BlockSpec((2, _R, DW), lambda i: (0, i, 0)),
            pl.BlockSpec((2, _R, 32), lambda i: (0, i, 0)),
            pl.BlockSpec((_R, 32), lambda i: (i, 0)),
            pl.BlockSpec((1, 32), lambda i: (0, 0)),
            pl.BlockSpec((32, 64), lambda i: (0, 0)),
        ],
        out_specs=pl.BlockSpec((_R, 64), lambda i: (i, 0)),
        out_shape=jax.ShapeDtypeStruct((NN, 64), jnp.float32),
    )(degp, p1, hs1, b1, w2)


def _head_body(degp, p2, hs2, b2, batchf, wfc, bfc, wfc2, bfc2, out,
               sums_sc, cnts_sc):
    i = pl.program_id(0)

    @pl.when(i == 0)
    def _init():
        sums_sc[...] = jnp.zeros_like(sums_sc)
        cnts_sc[...] = jnp.zeros_like(cnts_sc)

    dinv = _dinv_col(degp)
    h = (p2[0] + p2[1] + hs2[...]) * dinv + b2[...]
    h = jnp.maximum(h, 0.0)
    gids = lax.broadcasted_iota(jnp.int32, (1, GG), 1).astype(jnp.float32)
    onehot = jnp.where(batchf[...] == gids, 1.0, 0.0)
    sums_sc[...] += lax.dot_general(
        onehot, h, (((0,), (0,)), ((), ())),
        precision="highest", preferred_element_type=jnp.float32)
    ones_col = jnp.ones((_R, 1), jnp.float32)
    cnts_sc[...] += lax.dot_general(
        onehot, ones_col, (((0,), (0,)), ((), ())),
        precision="highest", preferred_element_type=jnp.float32)

    @pl.when(i == _GRID - 1)
    def _fin():
        pooled = sums_sc[...] / jnp.maximum(cnts_sc[...], 1.0)
        r = jnp.dot(pooled, wfc[...], preferred_element_type=jnp.float32) + bfc[...]
        r = jnp.maximum(r, 0.0)
        out[...] = jnp.dot(r, wfc2[...], preferred_element_type=jnp.float32) + bfc2[...]


def _head(degp, p2, hs2, b2, batchf, wfc, bfc, wfc2, bfc2):
    return pl.pallas_call(
        _head_body,
        grid=(_GRID,),
        in_specs=[
            pl.BlockSpec((2, _R, DW), lambda i: (0, i, 0)),
            pl.BlockSpec((2, _R, 64), lambda i: (0, i, 0)),
            pl.BlockSpec((_R, 64), lambda i: (i, 0)),
            pl.BlockSpec((1, 64), lambda i: (0, 0)),
            pl.BlockSpec((_R, 1), lambda i: (i, 0)),
            pl.BlockSpec((64, 8), lambda i: (0, 0)),
            pl.BlockSpec((1, 8), lambda i: (0, 0)),
            pl.BlockSpec((8, 1), lambda i: (0, 0)),
            pl.BlockSpec((1, 1), lambda i: (0, 0)),
        ],
        out_specs=pl.BlockSpec((GG, 1), lambda i: (0, 0)),
        out_shape=jax.ShapeDtypeStruct((GG, 1), jnp.float32),
        scratch_shapes=[
            pltpu.VMEM((GG, 64), jnp.float32),
            pltpu.VMEM((GG, 1), jnp.float32),
        ],
    )(degp, p2, hs2, b2, batchf, wfc, bfc, wfc2, bfc2)


def kernel(x, edge_index, batch, W1, b1, W2, b2, Wfc, bfc, Wfc2, bfc2):
    ei = edge_index.astype(jnp.int32).reshape(2, NW, NCH, CH)
    dst4 = edge_index[1].astype(jnp.int32).reshape(1, NW, NCH, CH)
    batchf = batch.astype(jnp.float32).reshape(NN, 1)
    ones16 = jnp.ones((CH, DW), jnp.float32)
    z16 = jnp.zeros((NN, DW), jnp.float32)
    z32 = jnp.zeros((NN, 32), jnp.float32)
    z64 = jnp.zeros((NN, 64), jnp.float32)

    degp = _deg_kernel(dst4, ones16, z16)
    h1 = _mm1(x, W1)
    hs1 = _scale1(degp, h1)
    p1 = _agg32(hs1, ei, z32)
    hs2 = _mid(degp, p1, hs1, b1.reshape(1, 32), W2)
    p2 = _agg64(hs2, ei, z64)
    out = _head(degp, p2, hs2, b2.reshape(1, 64), batchf,
                Wfc, bfc.reshape(1, 8), Wfc2, bfc2.reshape(1, 1))
    return out.reshape(-1)


# speedup vs baseline: 1.0497x; 1.0497x over previous
import functools

import jax
import jax.numpy as jnp
from jax import lax
from jax.experimental import pallas as pl
from jax.experimental.pallas import tpu as pltpu
from jax.experimental.pallas import tpu_sc as plsc

NN = 10000
EE = 320000
GG = 64
NW = 32
EPW = EE // NW
CH = 125
NCH = EPW // CH
RPS = 624
DW = 16
NBUF = 8
KPF = 6

_R = 2000
_GRID = NN // _R


def _sc_mesh():
    return plsc.VectorSubcoreMesh(core_axis_name="c", subcore_axis_name="s")


def _striped_copy(s, mk_src, mk_dst):
    base = pl.multiple_of(s * RPS, 8)

    @pl.when(s < 15)
    def _main():
        pltpu.sync_copy(mk_src(base, RPS), mk_dst(base, RPS))

    @pl.when(s == 15)
    def _tail():
        pltpu.sync_copy(mk_src(15 * RPS, NN - 15 * RPS),
                        mk_dst(15 * RPS, NN - 15 * RPS))


@functools.partial(
    pl.kernel,
    out_type=jax.ShapeDtypeStruct((2, NN, DW), jnp.float32),
    mesh=_sc_mesh(),
    scratch_types=[
        pltpu.VMEM((NCH, CH), jnp.int32),
        pltpu.VMEM((CH, DW), jnp.float32),
        pltpu.VMEM_SHARED((NN, DW), jnp.float32),
    ],
    compiler_params=pltpu.CompilerParams(use_tc_tiling_on_sc=False),
)
def _deg_kernel(ei, ones_hbm, zeros_hbm, out, idx_v, ones_v, deg_sh):
    c = lax.axis_index("c")
    s = lax.axis_index("s")
    w = c * 16 + s
    _striped_copy(s, lambda b, n: zeros_hbm.at[pl.ds(b, n)],
                  lambda b, n: deg_sh.at[pl.ds(b, n)])
    pltpu.sync_copy(ones_hbm, ones_v)
    pltpu.sync_copy(ei.at[1, w], idx_v)
    plsc.subcore_barrier()

    def step(j, carry):
        pltpu.sync_copy(ones_v, deg_sh.at[idx_v.at[j]], add=True)
        return carry

    lax.fori_loop(0, NCH, step, 0)
    plsc.subcore_barrier()
    _striped_copy(s, lambda b, n: deg_sh.at[pl.ds(b, n)],
                  lambda b, n: out.at[c, pl.ds(b, n)])


def _make_agg(F):
    @functools.partial(
        pl.kernel,
        out_type=jax.ShapeDtypeStruct((2, NN, F), jnp.float32),
        mesh=_sc_mesh(),
        scratch_types=[
            pltpu.VMEM((NCH, CH), jnp.int32),
            pltpu.VMEM((NCH, CH), jnp.int32),
            pltpu.VMEM((NBUF, CH, F), jnp.float32),
            pltpu.VMEM_SHARED((NN, F), jnp.float32),
            [pltpu.SemaphoreType.DMA] * NBUF,
            [pltpu.SemaphoreType.DMA] * NBUF,
        ],
        compiler_params=pltpu.CompilerParams(use_tc_tiling_on_sc=False),
    )
    def agg(hs, ei, zeros_hbm, out, srci_v, dsti_v, rows_v, agg_sh, gsems,
            ssems):
        c = lax.axis_index("c")
        s = lax.axis_index("s")
        w = c * 16 + s
        _striped_copy(s, lambda b, n: zeros_hbm.at[pl.ds(b, n)],
                      lambda b, n: agg_sh.at[pl.ds(b, n)])
        pltpu.sync_copy(ei.at[0, w], srci_v)
        pltpu.sync_copy(ei.at[1, w], dsti_v)
        plsc.subcore_barrier()

        for b in range(KPF):
            pltpu.async_copy(hs.at[srci_v.at[b]], rows_v.at[b], gsems[b])

        def outer(j0, carry):
            for r in range(NBUF):
                j = j0 * NBUF + r
                nb = (r + KPF) % NBUF
                pltpu.make_async_copy(
                    hs.at[srci_v.at[j]], rows_v.at[r], gsems[r]).wait()
                pltpu.async_copy(rows_v.at[r], agg_sh.at[dsti_v.at[j]],
                                 ssems[r], add=True)
                nxt = j + KPF

                @pl.when(nxt < NCH)
                def _prefetch():
                    @pl.when(j >= NBUF - KPF)
                    def _drain():
                        pltpu.make_async_copy(
                            rows_v.at[nb], agg_sh.at[dsti_v.at[0]],
                            ssems[nb]).wait()

                    pltpu.async_copy(hs.at[srci_v.at[nxt]], rows_v.at[nb],
                                     gsems[nb])
            return carry

        lax.fori_loop(0, NCH // NBUF, outer, 0)
        for b in range(NBUF):
            pltpu.make_async_copy(rows_v.at[b], agg_sh.at[dsti_v.at[0]],
                                  ssems[b]).wait()
        plsc.subcore_barrier()
        _striped_copy(s, lambda b, n: agg_sh.at[pl.ds(b, n)],
                      lambda b, n: out.at[c, pl.ds(b, n)])

    return agg


_agg32 = _make_agg(32)
_agg64 = _make_agg(64)


def _mm1_body(x, w1, h1):
    h1[...] = jnp.dot(x[...], w1[...], preferred_element_type=jnp.float32)


def _mm1(x, w1):
    return pl.pallas_call(
        _mm1_body,
        grid=(_GRID,),
        in_specs=[
            pl.BlockSpec((_R, 128), lambda i: (i, 0)),
            pl.BlockSpec((128, 32), lambda i: (0, 0)),
        ],
        out_specs=pl.BlockSpec((_R, 32), lambda i: (i, 0)),
        out_shape=jax.ShapeDtypeStruct((NN, 32), jnp.float32),
    )(x, w1)


def _dinv_col(degp):
    deg = degp[0, :, 0:1] + degp[1, :, 0:1] + 1.0
    return lax.rsqrt(deg)


def _scale1_body(degp, h1, hs1):
    hs1[...] = h1[...] * _dinv_col(degp)


def _scale1(degp, h1):
    return pl.pallas_call(
        _scale1_body,
        grid=(_GRID,),
        in_specs=[
            pl.BlockSpec((2, _R, DW), lambda i: (0, i, 0)),
            pl.BlockSpec((_R, 32), lambda i: (i, 0)),
        ],
        out_specs=pl.BlockSpec((_R, 32), lambda i: (i, 0)),
        out_shape=jax.ShapeDtypeStruct((NN, 32), jnp.float32),
    )(degp, h1)


def _mid_body(degp, p1, hs1, b1, w2, hs2):
    dinv = _dinv_col(degp)
    t = (p1[0] + p1[1] + hs1[...]) * dinv + b1[...]
    t = jnp.maximum(t, 0.0)
    h = jnp.dot(t, w2[...], preferred_element_type=jnp.float32)
    hs2[...] = h * dinv


def _mid(degp, p1, hs1, b1, w2):
    return pl.pallas_call(
        _mid_body,
        grid=(_GRID,),
        in_specs=[
            pl.BlockSpec((2, _R, DW), lambda i: (0, i, 0)),
            pl.BlockSpec((2, _R, 32), lambda i: (0, i, 0)),
            pl.BlockSpec((_R, 32), lambda i: (i, 0)),
            pl.BlockSpec((1, 32), lambda i: (0, 0)),
            pl.BlockSpec((32, 64), lambda i: (0, 0)),
        ],
        out_specs=pl.BlockSpec((_R, 64), lambda i: (i, 0)),
        out_shape=jax.ShapeDtypeStruct((NN, 64), jnp.float32),
    )(degp, p1, hs1, b1, w2)


def _head_body(degp, p2, hs2, b2, batchf, wfc, bfc, wfc2, bfc2, out,
               sums_sc, cnts_sc):
    i = pl.program_id(0)

    @pl.when(i == 0)
    def _init():
        sums_sc[...] = jnp.zeros_like(sums_sc)
        cnts_sc[...] = jnp.zeros_like(cnts_sc)

    dinv = _dinv_col(degp)
    h = (p2[0] + p2[1] + hs2[...]) * dinv + b2[...]
    h = jnp.maximum(h, 0.0)
    gids = lax.broadcasted_iota(jnp.int32, (1, GG), 1).astype(jnp.float32)
    onehot = jnp.where(batchf[...] == gids, 1.0, 0.0)
    sums_sc[...] += lax.dot_general(
        onehot, h, (((0,), (0,)), ((), ())),
        precision="highest", preferred_element_type=jnp.float32)
    ones_col = jnp.ones((_R, 1), jnp.float32)
    cnts_sc[...] += lax.dot_general(
        onehot, ones_col, (((0,), (0,)), ((), ())),
        precision="highest", preferred_element_type=jnp.float32)

    @pl.when(i == _GRID - 1)
    def _fin():
        pooled = sums_sc[...] / jnp.maximum(cnts_sc[...], 1.0)
        r = jnp.dot(pooled, wfc[...], preferred_element_type=jnp.float32) + bfc[...]
        r = jnp.maximum(r, 0.0)
        out[...] = jnp.dot(r, wfc2[...], preferred_element_type=jnp.float32) + bfc2[...]


def _head(degp, p2, hs2, b2, batchf, wfc, bfc, wfc2, bfc2):
    return pl.pallas_call(
        _head_body,
        grid=(_GRID,),
        in_specs=[
            pl.BlockSpec((2, _R, DW), lambda i: (0, i, 0)),
            pl.BlockSpec((2, _R, 64), lambda i: (0, i, 0)),
            pl.BlockSpec((_R, 64), lambda i: (i, 0)),
            pl.BlockSpec((1, 64), lambda i: (0, 0)),
            pl.BlockSpec((_R, 1), lambda i: (i, 0)),
            pl.BlockSpec((64, 8), lambda i: (0, 0)),
            pl.BlockSpec((1, 8), lambda i: (0, 0)),
            pl.BlockSpec((8, 1), lambda i: (0, 0)),
            pl.BlockSpec((1, 1), lambda i: (0, 0)),
        ],
        out_specs=pl.BlockSpec((GG, 1), lambda i: (0, 0)),
        out_shape=jax.ShapeDtypeStruct((GG, 1), jnp.float32),
        scratch_shapes=[
            pltpu.VMEM((GG, 64), jnp.float32),
            pltpu.VMEM((GG, 1), jnp.float32),
        ],
    )(degp, p2, hs2, b2, batchf, wfc, bfc, wfc2, bfc2)


def kernel(x, edge_index, batch, W1, b1, W2, b2, Wfc, bfc, Wfc2, bfc2):
    ei = edge_index.astype(jnp.int32).reshape(2, NW, NCH, CH)
    batchf = batch.astype(jnp.float32).reshape(NN, 1)
    ones16 = jnp.ones((CH, DW), jnp.float32)
    z16 = jnp.zeros((NN, DW), jnp.float32)
    z32 = jnp.zeros((NN, 32), jnp.float32)
    z64 = jnp.zeros((NN, 64), jnp.float32)

    degp = _deg_kernel(ei, ones16, z16)
    h1 = _mm1(x, W1)
    hs1 = _scale1(degp, h1)
    p1 = _agg32(hs1, ei, z32)
    hs2 = _mid(degp, p1, hs1, b1.reshape(1, 32), W2)
    p2 = _agg64(hs2, ei, z64)
    out = _head(degp, p2, hs2, b2.reshape(1, 64), batchf,
                Wfc, bfc.reshape(1, 8), Wfc2, bfc2.reshape(1, 1))
    return out.reshape(-1)
